# Initial kernel scaffold; baseline (speedup 1.0000x reference)
import jax
import jax.numpy as jnp
from jax.experimental import pallas as pl


def kernel(x, g0, g1, alpha, per_dim_scale, bias, input_perm, output_inv_perm):
    raise NotImplementedError("write your pallas kernel here")


# speedup vs baseline: 1.1185x; 1.1185x over previous
import functools

import jax
import jax.numpy as jnp
from jax.experimental import pallas as pl
from jax.experimental.pallas import tpu as pltpu

_N0, _N1 = 64, 64
_M0, _M1 = 64, 64
_R = 16
_BN = 256


def _mm_body(x_ref, v_ref, b_ref, o_ref):
    o_ref[...] = (
        jnp.dot(x_ref[...], v_ref[...], preferred_element_type=jnp.float32)
        + b_ref[...]
    )


@jax.jit
def _matmul_bias(x, v, bias2d):
    b, f_in = x.shape
    f_out = v.shape[1]
    return pl.pallas_call(
        _mm_body,
        grid=(f_out // _BN,),
        in_specs=[
            pl.BlockSpec((b, f_in), lambda n: (0, 0)),
            pl.BlockSpec((f_in, _BN), lambda n: (0, n)),
            pl.BlockSpec((1, _BN), lambda n: (0, n)),
        ],
        out_specs=pl.BlockSpec((b, _BN), lambda n: (0, n)),
        out_shape=jax.ShapeDtypeStruct((b, f_out), jnp.float32),
        compiler_params=pltpu.CompilerParams(
            dimension_semantics=("parallel",),
        ),
    )(x, v, bias2d)


def kernel(x, g0, g1, alpha, per_dim_scale, bias, input_perm, output_inv_perm):
    g0m = g0[0].reshape(_M0 * _N0, _R)
    g1m = g1[..., 0].reshape(_R, _M1 * _N1)
    t1 = g0m @ g1m
    w = (
        t1.reshape(_M0, _N0, _M1, _N1)
        .transpose(0, 2, 1, 3)
        .reshape(_M0 * _M1, _N0 * _N1)
    )
    a = w * (alpha * per_dim_scale)[:, None]
    v = a.T[jnp.argsort(input_perm)][:, output_inv_perm]
    return _matmul_bias(x, v, bias.reshape(1, -1))


# speedup vs baseline: 1.2242x; 1.0945x over previous
import functools

import jax
import jax.numpy as jnp
from jax.experimental import pallas as pl
from jax.experimental.pallas import tpu as pltpu

_N0, _N1 = 64, 64
_M0, _M1 = 64, 64
_R = 16
_BN = 256


def _mm_body(x_ref, v_ref, b_ref, o_ref):
    o_ref[...] = (
        jnp.dot(x_ref[...], v_ref[...], preferred_element_type=jnp.float32)
        + b_ref[...]
    )


@jax.jit
def _matmul_bias(x, v, bias2d):
    b, f_in = x.shape
    f_out = v.shape[1]
    return pl.pallas_call(
        _mm_body,
        grid=(f_out // _BN,),
        in_specs=[
            pl.BlockSpec((b, f_in), lambda n: (0, 0)),
            pl.BlockSpec((f_in, _BN), lambda n: (0, n)),
            pl.BlockSpec((1, _BN), lambda n: (0, n)),
        ],
        out_specs=pl.BlockSpec((b, _BN), lambda n: (0, n)),
        out_shape=jax.ShapeDtypeStruct((b, f_out), jnp.float32),
        compiler_params=pltpu.CompilerParams(
            dimension_semantics=("parallel",),
        ),
    )(x, v, bias2d)


def kernel(x, g0, g1, alpha, per_dim_scale, bias, input_perm, output_inv_perm):
    s4 = (alpha * per_dim_scale).reshape(_M0, _M1)
    v4 = jnp.einsum("ijr,rmk->jkim", g0[0], g1[..., 0]) * s4[None, None]
    v = v4.reshape(_N0 * _N1, _M0 * _M1)
    xp = x[:, input_perm]
    y = _matmul_bias(xp, v, jnp.zeros((1, v.shape[1]), jnp.float32))
    return y[:, output_inv_perm] + bias
